# Initial kernel scaffold; baseline (speedup 1.0000x reference)
import jax
import jax.numpy as jnp
from jax.experimental import pallas as pl


def kernel(x, edge_index, W1, b1, g1, be1, W2, b2, g2, be2, Wl, bl):
    raise NotImplementedError("write your pallas kernel here")


# speedup vs baseline: 29.6692x; 29.6692x over previous
import functools

import jax
import jax.numpy as jnp
from jax import lax
from jax.experimental import pallas as pl
from jax.experimental.pallas import tpu as pltpu
from jax.experimental.pallas import tpu_sc as plsc

N_NODES = 10000
N_PAD = 10240
PAD_ROWS = N_PAD - N_NODES
NC = 2
NS = 16
NW = NC * NS
CHUNK = 128
RPT = N_PAD // NS
EPS = 1e-5
ROW_BLK = 1280


def _worker(c, s):
    return c * NS + s


def _sc_degree(dst, ones, z, n_chunks):
    mesh = plsc.VectorSubcoreMesh(core_axis_name="c", subcore_axis_name="s")

    @functools.partial(
        pl.kernel,
        out_type=jax.ShapeDtypeStruct((NC, N_PAD, 8), jnp.float32),
        mesh=mesh,
        scratch_types=[
            pltpu.VMEM((n_chunks, CHUNK), jnp.int32),
            pltpu.VMEM((CHUNK, 8), jnp.float32),
            pltpu.VMEM_SHARED((N_PAD, 8), jnp.float32),
        ],
        compiler_params=pltpu.CompilerParams(use_tc_tiling_on_sc=False),
    )
    def deg_kernel(dst_hbm, ones_hbm, z_hbm, out_hbm, idx_v, ones_v, hist_sh):
        c = lax.axis_index("c")
        s = lax.axis_index("s")
        w = _worker(c, s)
        pltpu.sync_copy(dst_hbm.at[w], idx_v)
        pltpu.sync_copy(ones_hbm, ones_v)
        pltpu.sync_copy(z_hbm, hist_sh.at[pl.ds(s * RPT, RPT)])
        plsc.subcore_barrier()

        def body(j, carry):
            pltpu.sync_copy(ones_v, hist_sh.at[idx_v.at[j]], add=True)
            return carry

        lax.fori_loop(0, n_chunks, body, 0)
        plsc.subcore_barrier()
        pltpu.sync_copy(hist_sh.at[pl.ds(s * RPT, RPT)],
                        out_hbm.at[c, pl.ds(s * RPT, RPT)])

    return deg_kernel(dst, ones, z)


def _sc_aggregate(y, src, dst, z, n_chunks, d):
    mesh = plsc.VectorSubcoreMesh(core_axis_name="c", subcore_axis_name="s")

    @functools.partial(
        pl.kernel,
        out_type=jax.ShapeDtypeStruct((NC, N_PAD, d), jnp.float32),
        mesh=mesh,
        scratch_types=[
            pltpu.VMEM((n_chunks, CHUNK), jnp.int32),
            pltpu.VMEM((n_chunks, CHUNK), jnp.int32),
            pltpu.VMEM((CHUNK, d), jnp.float32),
            pltpu.VMEM_SHARED((N_PAD, d), jnp.float32),
            pltpu.SemaphoreType.DMA,
        ],
        compiler_params=pltpu.CompilerParams(use_tc_tiling_on_sc=False),
    )
    def agg_kernel(y_hbm, src_hbm, dst_hbm, z_hbm, out_hbm,
                   src_v, dst_v, rows_v, acc_sh, sem):
        c = lax.axis_index("c")
        s = lax.axis_index("s")
        w = _worker(c, s)
        pltpu.sync_copy(src_hbm.at[w], src_v)
        pltpu.sync_copy(dst_hbm.at[w], dst_v)
        pltpu.sync_copy(z_hbm, acc_sh.at[pl.ds(s * RPT, RPT)])
        plsc.subcore_barrier()

        def body(j, carry):
            pltpu.async_copy(y_hbm.at[src_v.at[j]], rows_v, sem).wait()
            pltpu.sync_copy(rows_v, acc_sh.at[dst_v.at[j]], add=True)
            return carry

        lax.fori_loop(0, n_chunks, body, 0)
        plsc.subcore_barrier()
        pltpu.sync_copy(acc_sh.at[pl.ds(s * RPT, RPT)],
                        out_hbm.at[c, pl.ds(s * RPT, RPT)])

    return agg_kernel(y, src, dst, z)


def _tc_prep(x, w1, degp):
    def body(degp_ref, x_ref, w_ref, y_ref, dinv_ref):
        deg = degp_ref[0, :, 0:1] + degp_ref[1, :, 0:1] + 1.0
        dinv = lax.rsqrt(deg)
        dinv_ref[...] = dinv
        y_ref[...] = jnp.dot(x_ref[...], w_ref[...],
                             preferred_element_type=jnp.float32) * dinv

    grid = N_PAD // ROW_BLK
    return pl.pallas_call(
        body,
        grid=(grid,),
        in_specs=[
            pl.BlockSpec((2, ROW_BLK, 8), lambda i: (0, i, 0)),
            pl.BlockSpec((ROW_BLK, 128), lambda i: (i, 0)),
            pl.BlockSpec((128, 64), lambda i: (0, 0)),
        ],
        out_specs=[
            pl.BlockSpec((ROW_BLK, 64), lambda i: (i, 0)),
            pl.BlockSpec((ROW_BLK, 1), lambda i: (i, 0)),
        ],
        out_shape=[
            jax.ShapeDtypeStruct((N_PAD, 64), jnp.float32),
            jax.ShapeDtypeStruct((N_PAD, 1), jnp.float32),
        ],
    )(degp, x, w1)


def _tc_mid(parts, y1, dinv, b1, g1, be1, w2):
    def body(p_ref, y_ref, dinv_ref, b_ref, g_ref, be_ref, w_ref, o_ref):
        dinv = dinv_ref[...]
        pre = (p_ref[0] + p_ref[1] + y_ref[...]) * dinv + b_ref[...]
        bns = g_ref[...] * (1.0 / jnp.sqrt(1.0 + EPS))
        h = jnp.maximum(pre * bns + be_ref[...], 0.0)
        o_ref[...] = jnp.dot(h, w_ref[...],
                             preferred_element_type=jnp.float32) * dinv

    grid = N_PAD // ROW_BLK
    return pl.pallas_call(
        body,
        grid=(grid,),
        in_specs=[
            pl.BlockSpec((2, ROW_BLK, 64), lambda i: (0, i, 0)),
            pl.BlockSpec((ROW_BLK, 64), lambda i: (i, 0)),
            pl.BlockSpec((ROW_BLK, 1), lambda i: (i, 0)),
            pl.BlockSpec((1, 64), lambda i: (0, 0)),
            pl.BlockSpec((1, 64), lambda i: (0, 0)),
            pl.BlockSpec((1, 64), lambda i: (0, 0)),
            pl.BlockSpec((64, 32), lambda i: (0, 0)),
        ],
        out_specs=pl.BlockSpec((ROW_BLK, 32), lambda i: (i, 0)),
        out_shape=jax.ShapeDtypeStruct((N_PAD, 32), jnp.float32),
    )(parts, y1, dinv, b1, g1, be1, w2)


def _tc_out(parts, y2, dinv, b2, g2, be2, wl, bl):
    def body(p_ref, y_ref, dinv_ref, b_ref, g_ref, be_ref, w_ref, bl_ref,
             o_ref):
        dinv = dinv_ref[...]
        pre = (p_ref[0] + p_ref[1] + y_ref[...]) * dinv + b_ref[...]
        bns = g_ref[...] * (1.0 / jnp.sqrt(1.0 + EPS))
        h = jnp.maximum(pre * bns + be_ref[...], 0.0)
        o_ref[...] = jnp.dot(h, w_ref[...],
                             preferred_element_type=jnp.float32) + bl_ref[...]

    grid = N_PAD // ROW_BLK
    return pl.pallas_call(
        body,
        grid=(grid,),
        in_specs=[
            pl.BlockSpec((2, ROW_BLK, 32), lambda i: (0, i, 0)),
            pl.BlockSpec((ROW_BLK, 32), lambda i: (i, 0)),
            pl.BlockSpec((ROW_BLK, 1), lambda i: (i, 0)),
            pl.BlockSpec((1, 32), lambda i: (0, 0)),
            pl.BlockSpec((1, 32), lambda i: (0, 0)),
            pl.BlockSpec((1, 32), lambda i: (0, 0)),
            pl.BlockSpec((32, 2), lambda i: (0, 0)),
            pl.BlockSpec((1, 2), lambda i: (0, 0)),
        ],
        out_specs=pl.BlockSpec((ROW_BLK, 2), lambda i: (i, 0)),
        out_shape=jax.ShapeDtypeStruct((N_PAD, 2), jnp.float32),
    )(parts, y2, dinv, b2, g2, be2, wl, bl)


def kernel(x, edge_index, W1, b1, g1, be1, W2, b2, g2, be2, Wl, bl):
    src = edge_index[0].astype(jnp.int32)
    dst = edge_index[1].astype(jnp.int32)
    e = src.shape[0]
    n_chunks = -(-e // (NW * CHUNK))
    epad = NW * n_chunks * CHUNK
    pad_n = epad - e
    pad_idx = N_NODES + (jnp.arange(pad_n, dtype=jnp.int32) % PAD_ROWS)
    srcp = jnp.concatenate([src, pad_idx]).reshape(NW, n_chunks, CHUNK)
    dstp = jnp.concatenate([dst, pad_idx]).reshape(NW, n_chunks, CHUNK)

    xp = jnp.zeros((N_PAD, 128), jnp.float32).at[:N_NODES].set(x)
    ones8 = jnp.ones((CHUNK, 8), jnp.float32)
    z8 = jnp.zeros((RPT, 8), jnp.float32)
    z64 = jnp.zeros((RPT, 64), jnp.float32)
    z32 = jnp.zeros((RPT, 32), jnp.float32)

    degp = _sc_degree(dstp, ones8, z8, n_chunks)
    y1, dinv = _tc_prep(xp, W1, degp)
    parts1 = _sc_aggregate(y1, srcp, dstp, z64, n_chunks, 64)
    y2 = _tc_mid(parts1, y1, dinv, b1.reshape(1, 64), g1.reshape(1, 64),
                 be1.reshape(1, 64), W2)
    parts2 = _sc_aggregate(y2, srcp, dstp, z32, n_chunks, 32)
    out = _tc_out(parts2, y2, dinv, b2.reshape(1, 32), g2.reshape(1, 32),
                  be2.reshape(1, 32), Wl, bl.reshape(1, 2))
    return out[:N_NODES]


# speedup vs baseline: 46.1683x; 1.5561x over previous
import functools

import jax
import jax.numpy as jnp
from jax import lax
from jax.experimental import pallas as pl
from jax.experimental.pallas import tpu as pltpu
from jax.experimental.pallas import tpu_sc as plsc

N_NODES = 10000
N_PAD = 10240
PAD_ROWS = N_PAD - N_NODES
NC = 2
NS = 16
NW = NC * NS
CHUNK = 128
RPT = N_PAD // NS
EPS = 1e-5
ROW_BLK = 1280


def _worker(c, s):
    return c * NS + s


def _sc_degree(dst, ones, z, n_chunks):
    mesh = plsc.VectorSubcoreMesh(core_axis_name="c", subcore_axis_name="s")

    @functools.partial(
        pl.kernel,
        out_type=jax.ShapeDtypeStruct((NC, N_PAD, 8), jnp.float32),
        mesh=mesh,
        scratch_types=[
            pltpu.VMEM((n_chunks, CHUNK), jnp.int32),
            pltpu.VMEM((CHUNK, 8), jnp.float32),
            pltpu.VMEM_SHARED((N_PAD, 8), jnp.float32),
        ],
        compiler_params=pltpu.CompilerParams(use_tc_tiling_on_sc=False),
    )
    def deg_kernel(dst_hbm, ones_hbm, z_hbm, out_hbm, idx_v, ones_v, hist_sh):
        c = lax.axis_index("c")
        s = lax.axis_index("s")
        w = _worker(c, s)
        pltpu.sync_copy(dst_hbm.at[w], idx_v)
        pltpu.sync_copy(ones_hbm, ones_v)
        pltpu.sync_copy(z_hbm, hist_sh.at[pl.ds(s * RPT, RPT)])
        plsc.subcore_barrier()

        def body(j, carry):
            pltpu.sync_copy(ones_v, hist_sh.at[idx_v.at[j]], add=True)
            return carry

        lax.fori_loop(0, n_chunks, body, 0)
        plsc.subcore_barrier()
        pltpu.sync_copy(hist_sh.at[pl.ds(s * RPT, RPT)],
                        out_hbm.at[c, pl.ds(s * RPT, RPT)])

    return deg_kernel(dst, ones, z)


NBUF = 4


def _sc_aggregate(y, src, dst, z, n_chunks, d):
    mesh = plsc.VectorSubcoreMesh(core_axis_name="c", subcore_axis_name="s")

    @functools.partial(
        pl.kernel,
        out_type=jax.ShapeDtypeStruct((NC, N_PAD, d), jnp.float32),
        mesh=mesh,
        scratch_types=[
            pltpu.VMEM((n_chunks, CHUNK), jnp.int32),
            pltpu.VMEM((n_chunks, CHUNK), jnp.int32),
            pltpu.VMEM((NBUF, CHUNK, d), jnp.float32),
            pltpu.VMEM_SHARED((N_PAD, d), jnp.float32),
        ] + [pltpu.SemaphoreType.DMA] * NBUF,
        compiler_params=pltpu.CompilerParams(use_tc_tiling_on_sc=False),
    )
    def agg_kernel(y_hbm, src_hbm, dst_hbm, z_hbm, out_hbm,
                   src_v, dst_v, rows_v, acc_sh, *sems):
        c = lax.axis_index("c")
        s = lax.axis_index("s")
        w = _worker(c, s)
        pltpu.sync_copy(src_hbm.at[w], src_v)
        pltpu.sync_copy(dst_hbm.at[w], dst_v)
        for b in range(NBUF):
            pltpu.async_copy(y_hbm.at[src_v.at[b]], rows_v.at[b], sems[b])
        pltpu.sync_copy(z_hbm, acc_sh.at[pl.ds(s * RPT, RPT)])
        plsc.subcore_barrier()

        def body(g, carry):
            base = g * NBUF
            for b in range(NBUF):
                j = base + b
                pltpu.make_async_copy(y_hbm.at[src_v.at[j]], rows_v.at[b],
                                      sems[b]).wait()
                pltpu.sync_copy(rows_v.at[b], acc_sh.at[dst_v.at[j]],
                                add=True)
                pltpu.async_copy(y_hbm.at[src_v.at[j + NBUF]], rows_v.at[b],
                                 sems[b])
            return carry

        lax.fori_loop(0, n_chunks // NBUF - 1, body, 0)
        for b in range(NBUF):
            j = n_chunks - NBUF + b
            pltpu.make_async_copy(y_hbm.at[src_v.at[j]], rows_v.at[b],
                                  sems[b]).wait()
            pltpu.sync_copy(rows_v.at[b], acc_sh.at[dst_v.at[j]], add=True)
        plsc.subcore_barrier()
        pltpu.sync_copy(acc_sh.at[pl.ds(s * RPT, RPT)],
                        out_hbm.at[c, pl.ds(s * RPT, RPT)])

    return agg_kernel(y, src, dst, z)


def _tc_prep(x, w1, degp):
    def body(degp_ref, x_ref, w_ref, y_ref, dinv_ref):
        deg = degp_ref[0, :, 0:1] + degp_ref[1, :, 0:1] + 1.0
        dinv = lax.rsqrt(deg)
        dinv_ref[...] = dinv
        y_ref[...] = jnp.dot(x_ref[...], w_ref[...],
                             preferred_element_type=jnp.float32) * dinv

    grid = N_PAD // ROW_BLK
    return pl.pallas_call(
        body,
        grid=(grid,),
        in_specs=[
            pl.BlockSpec((2, ROW_BLK, 8), lambda i: (0, i, 0)),
            pl.BlockSpec((ROW_BLK, 128), lambda i: (i, 0)),
            pl.BlockSpec((128, 64), lambda i: (0, 0)),
        ],
        out_specs=[
            pl.BlockSpec((ROW_BLK, 64), lambda i: (i, 0)),
            pl.BlockSpec((ROW_BLK, 1), lambda i: (i, 0)),
        ],
        out_shape=[
            jax.ShapeDtypeStruct((N_PAD, 64), jnp.float32),
            jax.ShapeDtypeStruct((N_PAD, 1), jnp.float32),
        ],
    )(degp, x, w1)


def _tc_mid(parts, y1, dinv, b1, g1, be1, w2):
    def body(p_ref, y_ref, dinv_ref, b_ref, g_ref, be_ref, w_ref, o_ref):
        dinv = dinv_ref[...]
        pre = (p_ref[0] + p_ref[1] + y_ref[...]) * dinv + b_ref[...]
        bns = g_ref[...] * (1.0 / jnp.sqrt(1.0 + EPS))
        h = jnp.maximum(pre * bns + be_ref[...], 0.0)
        o_ref[...] = jnp.dot(h, w_ref[...],
                             preferred_element_type=jnp.float32) * dinv

    grid = N_PAD // ROW_BLK
    return pl.pallas_call(
        body,
        grid=(grid,),
        in_specs=[
            pl.BlockSpec((2, ROW_BLK, 64), lambda i: (0, i, 0)),
            pl.BlockSpec((ROW_BLK, 64), lambda i: (i, 0)),
            pl.BlockSpec((ROW_BLK, 1), lambda i: (i, 0)),
            pl.BlockSpec((1, 64), lambda i: (0, 0)),
            pl.BlockSpec((1, 64), lambda i: (0, 0)),
            pl.BlockSpec((1, 64), lambda i: (0, 0)),
            pl.BlockSpec((64, 32), lambda i: (0, 0)),
        ],
        out_specs=pl.BlockSpec((ROW_BLK, 32), lambda i: (i, 0)),
        out_shape=jax.ShapeDtypeStruct((N_PAD, 32), jnp.float32),
    )(parts, y1, dinv, b1, g1, be1, w2)


def _tc_out(parts, y2, dinv, b2, g2, be2, wl, bl):
    def body(p_ref, y_ref, dinv_ref, b_ref, g_ref, be_ref, w_ref, bl_ref,
             o_ref):
        dinv = dinv_ref[...]
        pre = (p_ref[0] + p_ref[1] + y_ref[...]) * dinv + b_ref[...]
        bns = g_ref[...] * (1.0 / jnp.sqrt(1.0 + EPS))
        h = jnp.maximum(pre * bns + be_ref[...], 0.0)
        o_ref[...] = jnp.dot(h, w_ref[...],
                             preferred_element_type=jnp.float32) + bl_ref[...]

    grid = N_PAD // ROW_BLK
    return pl.pallas_call(
        body,
        grid=(grid,),
        in_specs=[
            pl.BlockSpec((2, ROW_BLK, 32), lambda i: (0, i, 0)),
            pl.BlockSpec((ROW_BLK, 32), lambda i: (i, 0)),
            pl.BlockSpec((ROW_BLK, 1), lambda i: (i, 0)),
            pl.BlockSpec((1, 32), lambda i: (0, 0)),
            pl.BlockSpec((1, 32), lambda i: (0, 0)),
            pl.BlockSpec((1, 32), lambda i: (0, 0)),
            pl.BlockSpec((32, 2), lambda i: (0, 0)),
            pl.BlockSpec((1, 2), lambda i: (0, 0)),
        ],
        out_specs=pl.BlockSpec((ROW_BLK, 2), lambda i: (i, 0)),
        out_shape=jax.ShapeDtypeStruct((N_PAD, 2), jnp.float32),
    )(parts, y2, dinv, b2, g2, be2, wl, bl)


def kernel(x, edge_index, W1, b1, g1, be1, W2, b2, g2, be2, Wl, bl):
    src = edge_index[0].astype(jnp.int32)
    dst = edge_index[1].astype(jnp.int32)
    e = src.shape[0]
    n_chunks = -(-e // (NW * CHUNK))
    n_chunks = -(-n_chunks // NBUF) * NBUF
    epad = NW * n_chunks * CHUNK
    pad_n = epad - e
    pad_idx = N_NODES + (jnp.arange(pad_n, dtype=jnp.int32) % PAD_ROWS)
    srcp = jnp.concatenate([src, pad_idx]).reshape(NW, n_chunks, CHUNK)
    dstp = jnp.concatenate([dst, pad_idx]).reshape(NW, n_chunks, CHUNK)

    xp = jnp.zeros((N_PAD, 128), jnp.float32).at[:N_NODES].set(x)
    ones8 = jnp.ones((CHUNK, 8), jnp.float32)
    z8 = jnp.zeros((RPT, 8), jnp.float32)
    z64 = jnp.zeros((RPT, 64), jnp.float32)
    z32 = jnp.zeros((RPT, 32), jnp.float32)

    degp = _sc_degree(dstp, ones8, z8, n_chunks)
    y1, dinv = _tc_prep(xp, W1, degp)
    parts1 = _sc_aggregate(y1, srcp, dstp, z64, n_chunks, 64)
    y2 = _tc_mid(parts1, y1, dinv, b1.reshape(1, 64), g1.reshape(1, 64),
                 be1.reshape(1, 64), W2)
    parts2 = _sc_aggregate(y2, srcp, dstp, z32, n_chunks, 32)
    out = _tc_out(parts2, y2, dinv, b2.reshape(1, 32), g2.reshape(1, 32),
                  be2.reshape(1, 32), Wl, bl.reshape(1, 2))
    return out[:N_NODES]


# speedup vs baseline: 47.4373x; 1.0275x over previous
import functools

import jax
import jax.numpy as jnp
from jax import lax
from jax.experimental import pallas as pl
from jax.experimental.pallas import tpu as pltpu
from jax.experimental.pallas import tpu_sc as plsc

N_NODES = 10000
N_PAD = 10240
PAD_ROWS = N_PAD - N_NODES
NC = 2
NS = 16
NW = NC * NS
CHUNK = 125
RPT = N_PAD // NS
EPS = 1e-5
ROW_BLK = 1280


def _worker(c, s):
    return c * NS + s


def _sc_degree(dst, ones, z, n_chunks):
    mesh = plsc.VectorSubcoreMesh(core_axis_name="c", subcore_axis_name="s")

    @functools.partial(
        pl.kernel,
        out_type=jax.ShapeDtypeStruct((NC, N_PAD, 8), jnp.float32),
        mesh=mesh,
        scratch_types=[
            pltpu.VMEM((n_chunks, CHUNK), jnp.int32),
            pltpu.VMEM((CHUNK, 8), jnp.float32),
            pltpu.VMEM_SHARED((N_PAD, 8), jnp.float32),
            pltpu.SemaphoreType.DMA,
        ],
        compiler_params=pltpu.CompilerParams(use_tc_tiling_on_sc=False),
    )
    def deg_kernel(dst_hbm, ones_hbm, z_hbm, out_hbm, idx_v, ones_v, hist_sh,
                   sem):
        c = lax.axis_index("c")
        s = lax.axis_index("s")
        w = _worker(c, s)
        pltpu.sync_copy(dst_hbm.at[w], idx_v)
        pltpu.sync_copy(ones_hbm, ones_v)
        pltpu.sync_copy(z_hbm, hist_sh.at[pl.ds(s * RPT, RPT)])
        plsc.subcore_barrier()

        def body(j, carry):
            pltpu.async_copy(ones_v, hist_sh.at[idx_v.at[j]], sem, add=True)
            return carry

        lax.fori_loop(0, n_chunks, body, 0)

        def drain(j, carry):
            pltpu.make_async_copy(ones_v, hist_sh.at[idx_v.at[0]], sem).wait()
            return carry

        lax.fori_loop(0, n_chunks, drain, 0)
        plsc.subcore_barrier()
        pltpu.sync_copy(hist_sh.at[pl.ds(s * RPT, RPT)],
                        out_hbm.at[c, pl.ds(s * RPT, RPT)])

    return deg_kernel(dst, ones, z)


NBUF = 8


def _sc_aggregate(y, src, dst, z, n_chunks, d):
    mesh = plsc.VectorSubcoreMesh(core_axis_name="c", subcore_axis_name="s")

    @functools.partial(
        pl.kernel,
        out_type=jax.ShapeDtypeStruct((NC, N_PAD, d), jnp.float32),
        mesh=mesh,
        scratch_types=[
            pltpu.VMEM((n_chunks, CHUNK), jnp.int32),
            pltpu.VMEM((n_chunks, CHUNK), jnp.int32),
            pltpu.VMEM((NBUF, CHUNK, d), jnp.float32),
            pltpu.VMEM_SHARED((N_PAD, d), jnp.float32),
        ] + [pltpu.SemaphoreType.DMA] * NBUF,
        compiler_params=pltpu.CompilerParams(use_tc_tiling_on_sc=False),
    )
    def agg_kernel(y_hbm, src_hbm, dst_hbm, z_hbm, out_hbm,
                   src_v, dst_v, rows_v, acc_sh, *sems):
        c = lax.axis_index("c")
        s = lax.axis_index("s")
        w = _worker(c, s)
        pltpu.sync_copy(src_hbm.at[w], src_v)
        pltpu.sync_copy(dst_hbm.at[w], dst_v)
        for b in range(NBUF):
            pltpu.async_copy(y_hbm.at[src_v.at[b]], rows_v.at[b], sems[b])
        pltpu.sync_copy(z_hbm, acc_sh.at[pl.ds(s * RPT, RPT)])
        plsc.subcore_barrier()

        def body(g, carry):
            base = g * NBUF
            for b in range(NBUF):
                j = base + b
                pltpu.make_async_copy(y_hbm.at[src_v.at[j]], rows_v.at[b],
                                      sems[b]).wait()
                pltpu.sync_copy(rows_v.at[b], acc_sh.at[dst_v.at[j]],
                                add=True)
                pltpu.async_copy(y_hbm.at[src_v.at[j + NBUF]], rows_v.at[b],
                                 sems[b])
            return carry

        lax.fori_loop(0, n_chunks // NBUF - 1, body, 0)
        for b in range(NBUF):
            j = n_chunks - NBUF + b
            pltpu.make_async_copy(y_hbm.at[src_v.at[j]], rows_v.at[b],
                                  sems[b]).wait()
            pltpu.sync_copy(rows_v.at[b], acc_sh.at[dst_v.at[j]], add=True)
        plsc.subcore_barrier()
        pltpu.sync_copy(acc_sh.at[pl.ds(s * RPT, RPT)],
                        out_hbm.at[c, pl.ds(s * RPT, RPT)])

    return agg_kernel(y, src, dst, z)


def _tc_prep(x, w1, degp):
    def body(degp_ref, x_ref, w_ref, y_ref, dinv_ref):
        deg = degp_ref[0, :, 0:1] + degp_ref[1, :, 0:1] + 1.0
        dinv = lax.rsqrt(deg)
        dinv_ref[...] = dinv
        y_ref[...] = jnp.dot(x_ref[...], w_ref[...],
                             preferred_element_type=jnp.float32) * dinv

    grid = N_PAD // ROW_BLK
    return pl.pallas_call(
        body,
        grid=(grid,),
        in_specs=[
            pl.BlockSpec((2, ROW_BLK, 8), lambda i: (0, i, 0)),
            pl.BlockSpec((ROW_BLK, 128), lambda i: (i, 0)),
            pl.BlockSpec((128, 64), lambda i: (0, 0)),
        ],
        out_specs=[
            pl.BlockSpec((ROW_BLK, 64), lambda i: (i, 0)),
            pl.BlockSpec((ROW_BLK, 1), lambda i: (i, 0)),
        ],
        out_shape=[
            jax.ShapeDtypeStruct((N_PAD, 64), jnp.float32),
            jax.ShapeDtypeStruct((N_PAD, 1), jnp.float32),
        ],
    )(degp, x, w1)


def _tc_mid(parts, y1, dinv, b1, g1, be1, w2):
    def body(p_ref, y_ref, dinv_ref, b_ref, g_ref, be_ref, w_ref, o_ref):
        dinv = dinv_ref[...]
        pre = (p_ref[0] + p_ref[1] + y_ref[...]) * dinv + b_ref[...]
        bns = g_ref[...] * (1.0 / jnp.sqrt(1.0 + EPS))
        h = jnp.maximum(pre * bns + be_ref[...], 0.0)
        o_ref[...] = jnp.dot(h, w_ref[...],
                             preferred_element_type=jnp.float32) * dinv

    grid = N_PAD // ROW_BLK
    return pl.pallas_call(
        body,
        grid=(grid,),
        in_specs=[
            pl.BlockSpec((2, ROW_BLK, 64), lambda i: (0, i, 0)),
            pl.BlockSpec((ROW_BLK, 64), lambda i: (i, 0)),
            pl.BlockSpec((ROW_BLK, 1), lambda i: (i, 0)),
            pl.BlockSpec((1, 64), lambda i: (0, 0)),
            pl.BlockSpec((1, 64), lambda i: (0, 0)),
            pl.BlockSpec((1, 64), lambda i: (0, 0)),
            pl.BlockSpec((64, 32), lambda i: (0, 0)),
        ],
        out_specs=pl.BlockSpec((ROW_BLK, 32), lambda i: (i, 0)),
        out_shape=jax.ShapeDtypeStruct((N_PAD, 32), jnp.float32),
    )(parts, y1, dinv, b1, g1, be1, w2)


def _tc_out(parts, y2, dinv, b2, g2, be2, wl, bl):
    def body(p_ref, y_ref, dinv_ref, b_ref, g_ref, be_ref, w_ref, bl_ref,
             o_ref):
        dinv = dinv_ref[...]
        pre = (p_ref[0] + p_ref[1] + y_ref[...]) * dinv + b_ref[...]
        bns = g_ref[...] * (1.0 / jnp.sqrt(1.0 + EPS))
        h = jnp.maximum(pre * bns + be_ref[...], 0.0)
        o_ref[...] = jnp.dot(h, w_ref[...],
                             preferred_element_type=jnp.float32) + bl_ref[...]

    grid = N_PAD // ROW_BLK
    return pl.pallas_call(
        body,
        grid=(grid,),
        in_specs=[
            pl.BlockSpec((2, ROW_BLK, 32), lambda i: (0, i, 0)),
            pl.BlockSpec((ROW_BLK, 32), lambda i: (i, 0)),
            pl.BlockSpec((ROW_BLK, 1), lambda i: (i, 0)),
            pl.BlockSpec((1, 32), lambda i: (0, 0)),
            pl.BlockSpec((1, 32), lambda i: (0, 0)),
            pl.BlockSpec((1, 32), lambda i: (0, 0)),
            pl.BlockSpec((32, 2), lambda i: (0, 0)),
            pl.BlockSpec((1, 2), lambda i: (0, 0)),
        ],
        out_specs=pl.BlockSpec((ROW_BLK, 2), lambda i: (i, 0)),
        out_shape=jax.ShapeDtypeStruct((N_PAD, 2), jnp.float32),
    )(parts, y2, dinv, b2, g2, be2, wl, bl)


def kernel(x, edge_index, W1, b1, g1, be1, W2, b2, g2, be2, Wl, bl):
    src = edge_index[0].astype(jnp.int32)
    dst = edge_index[1].astype(jnp.int32)
    e = src.shape[0]
    n_chunks = -(-e // (NW * CHUNK))
    n_chunks = -(-n_chunks // NBUF) * NBUF
    epad = NW * n_chunks * CHUNK
    pad_n = epad - e
    if pad_n:
        pad_idx = N_NODES + (jnp.arange(pad_n, dtype=jnp.int32) % PAD_ROWS)
        src = jnp.concatenate([src, pad_idx])
        dst = jnp.concatenate([dst, pad_idx])
    srcp = src.reshape(NW, n_chunks, CHUNK)
    dstp = dst.reshape(NW, n_chunks, CHUNK)

    xp = jnp.zeros((N_PAD, 128), jnp.float32).at[:N_NODES].set(x)
    ones8 = jnp.ones((CHUNK, 8), jnp.float32)
    z8 = jnp.zeros((RPT, 8), jnp.float32)
    z64 = jnp.zeros((RPT, 64), jnp.float32)
    z32 = jnp.zeros((RPT, 32), jnp.float32)

    degp = _sc_degree(dstp, ones8, z8, n_chunks)
    y1, dinv = _tc_prep(xp, W1, degp)
    parts1 = _sc_aggregate(y1, srcp, dstp, z64, n_chunks, 64)
    y2 = _tc_mid(parts1, y1, dinv, b1.reshape(1, 64), g1.reshape(1, 64),
                 be1.reshape(1, 64), W2)
    parts2 = _sc_aggregate(y2, srcp, dstp, z32, n_chunks, 32)
    out = _tc_out(parts2, y2, dinv, b2.reshape(1, 32), g2.reshape(1, 32),
                  be2.reshape(1, 32), Wl, bl.reshape(1, 2))
    return out[:N_NODES]


# speedup vs baseline: 53.4219x; 1.1262x over previous
import functools

import jax
import jax.numpy as jnp
from jax import lax
from jax.experimental import pallas as pl
from jax.experimental.pallas import tpu as pltpu
from jax.experimental.pallas import tpu_sc as plsc

N_NODES = 10000
N_PAD = 10240
PAD_ROWS = N_PAD - N_NODES
NC = 2
NS = 16
NW = NC * NS
CHUNK = 125
RPT = N_PAD // NS
EPS = 1e-5
ROW_BLK = 1280
NBUF = 8


def _worker(c, s):
    return c * NS + s


def _sc_degree(dst, ones, z, n_chunks):
    mesh = plsc.VectorSubcoreMesh(core_axis_name="c", subcore_axis_name="s")

    @functools.partial(
        pl.kernel,
        out_type=jax.ShapeDtypeStruct((NC, N_PAD, 8), jnp.float32),
        mesh=mesh,
        scratch_types=[
            pltpu.VMEM((n_chunks, CHUNK), jnp.int32),
            pltpu.VMEM((CHUNK, 8), jnp.float32),
            pltpu.VMEM_SHARED((N_PAD, 8), jnp.float32),
            pltpu.SemaphoreType.DMA,
        ],
        compiler_params=pltpu.CompilerParams(use_tc_tiling_on_sc=False),
    )
    def deg_kernel(dst_hbm, ones_hbm, z_hbm, out_hbm, idx_v, ones_v, hist_sh,
                   sem):
        c = lax.axis_index("c")
        s = lax.axis_index("s")
        w = _worker(c, s)
        pltpu.sync_copy(dst_hbm.at[w], idx_v)
        pltpu.sync_copy(ones_hbm, ones_v)
        pltpu.sync_copy(z_hbm, hist_sh.at[pl.ds(s * RPT, RPT)])
        plsc.subcore_barrier()

        def body(j, carry):
            pltpu.async_copy(ones_v, hist_sh.at[idx_v.at[j]], sem, add=True)
            return carry

        lax.fori_loop(0, n_chunks, body, 0)

        def drain(j, carry):
            pltpu.make_async_copy(ones_v, hist_sh.at[idx_v.at[0]], sem).wait()
            return carry

        lax.fori_loop(0, n_chunks, drain, 0)
        plsc.subcore_barrier()
        pltpu.sync_copy(hist_sh.at[pl.ds(s * RPT, RPT)],
                        out_hbm.at[c, pl.ds(s * RPT, RPT)])

    return deg_kernel(dst, ones, z)


def _sc_aggregate(yp, src, dst, z, n_chunks, d):
    mesh = plsc.VectorSubcoreMesh(core_axis_name="c", subcore_axis_name="s")

    @functools.partial(
        pl.kernel,
        out_type=jax.ShapeDtypeStruct((NC, N_PAD, d), jnp.float32),
        mesh=mesh,
        scratch_types=[
            pltpu.VMEM((n_chunks, CHUNK), jnp.int32),
            pltpu.VMEM((n_chunks, CHUNK), jnp.int32),
            pltpu.VMEM((NBUF, CHUNK, d), jnp.float32),
            pltpu.VMEM_SHARED((N_PAD, d), jnp.float32),
        ] + [pltpu.SemaphoreType.DMA] * NBUF,
        compiler_params=pltpu.CompilerParams(use_tc_tiling_on_sc=False),
    )
    def agg_kernel(y_hbm, src_hbm, dst_hbm, z_hbm, out_hbm,
                   src_v, dst_v, rows_v, acc_sh, *sems):
        c = lax.axis_index("c")
        s = lax.axis_index("s")
        w = _worker(c, s)
        y_flat = y_hbm
        pltpu.sync_copy(src_hbm.at[w], src_v)
        pltpu.sync_copy(dst_hbm.at[w], dst_v)
        for b in range(NBUF):
            pltpu.async_copy(y_flat.at[src_v.at[b]], rows_v.at[b], sems[b])
        pltpu.sync_copy(z_hbm, acc_sh.at[pl.ds(s * RPT, RPT)])
        plsc.subcore_barrier()

        def body(g, carry):
            base = g * NBUF
            for b in range(NBUF):
                j = base + b
                pltpu.make_async_copy(y_flat.at[src_v.at[j]], rows_v.at[b],
                                      sems[b]).wait()
                pltpu.sync_copy(rows_v.at[b], acc_sh.at[dst_v.at[j]],
                                add=True)
                pltpu.async_copy(y_flat.at[src_v.at[j + NBUF]], rows_v.at[b],
                                 sems[b])
            return carry

        lax.fori_loop(0, n_chunks // NBUF - 1, body, 0)
        for b in range(NBUF):
            j = n_chunks - NBUF + b
            pltpu.make_async_copy(y_flat.at[src_v.at[j]], rows_v.at[b],
                                  sems[b]).wait()
            pltpu.sync_copy(rows_v.at[b], acc_sh.at[dst_v.at[j]], add=True)
        plsc.subcore_barrier()
        pltpu.sync_copy(acc_sh.at[pl.ds(s * RPT, RPT)],
                        out_hbm.at[c, pl.ds(s * RPT, RPT)])

    return agg_kernel(yp.reshape(N_PAD, d), src, dst, z)


def _tc_prep(x2, w1b, dinvp1):
    def body(x_ref, w_ref, d_ref, y_ref):
        y_ref[...] = jnp.dot(x_ref[...], w_ref[...],
                             preferred_element_type=jnp.float32) * d_ref[...]

    grid = N_PAD // ROW_BLK
    rb = ROW_BLK // 2
    return pl.pallas_call(
        body,
        grid=(grid,),
        in_specs=[
            pl.BlockSpec((rb, 256), lambda i: (i, 0)),
            pl.BlockSpec((256, 128), lambda i: (0, 0)),
            pl.BlockSpec((rb, 128), lambda i: (i, 0)),
        ],
        out_specs=pl.BlockSpec((rb, 128), lambda i: (i, 0)),
        out_shape=jax.ShapeDtypeStruct((N_PAD // 2, 128), jnp.float32),
    )(x2, w1b, dinvp1)


def _tc_mid(parts, y1p, dinvp1, dinvp2, b1p, g1p, be1p, w2b4):
    def body(p_ref, y_ref, d1_ref, d2_ref, b_ref, g_ref, be_ref, w_ref,
             o_ref):
        pre = (p_ref[0] + p_ref[1] + y_ref[...]) * d1_ref[...] + b_ref[...]
        bns = g_ref[...] * (1.0 / jnp.sqrt(1.0 + EPS))
        h = jnp.maximum(pre * bns + be_ref[...], 0.0)
        hm = h.reshape(h.shape[0] // 2, 256)
        o_ref[...] = jnp.dot(hm, w_ref[...],
                             preferred_element_type=jnp.float32) * d2_ref[...]

    grid = N_PAD // ROW_BLK
    rb = ROW_BLK // 2
    rq = ROW_BLK // 4
    return pl.pallas_call(
        body,
        grid=(grid,),
        in_specs=[
            pl.BlockSpec((2, rb, 128), lambda i: (0, i, 0)),
            pl.BlockSpec((rb, 128), lambda i: (i, 0)),
            pl.BlockSpec((rb, 128), lambda i: (i, 0)),
            pl.BlockSpec((rq, 128), lambda i: (i, 0)),
            pl.BlockSpec((1, 128), lambda i: (0, 0)),
            pl.BlockSpec((1, 128), lambda i: (0, 0)),
            pl.BlockSpec((1, 128), lambda i: (0, 0)),
            pl.BlockSpec((256, 128), lambda i: (0, 0)),
        ],
        out_specs=pl.BlockSpec((rq, 128), lambda i: (i, 0)),
        out_shape=jax.ShapeDtypeStruct((N_PAD // 4, 128), jnp.float32),
    )(parts, y1p, dinvp1, dinvp2, b1p, g1p, be1p, w2b4)


def _tc_out(parts, y2p, dinvp2, b2p, g2p, be2p, wlq, blq):
    def body(p_ref, y_ref, d_ref, b_ref, g_ref, be_ref, w_ref, bl_ref,
             o_ref):
        pre = (p_ref[0] + p_ref[1] + y_ref[...]) * d_ref[...] + b_ref[...]
        bns = g_ref[...] * (1.0 / jnp.sqrt(1.0 + EPS))
        h = jnp.maximum(pre * bns + be_ref[...], 0.0)
        o_ref[...] = jnp.dot(h, w_ref[...],
                             preferred_element_type=jnp.float32) + bl_ref[...]

    grid = N_PAD // ROW_BLK
    rq = ROW_BLK // 4
    return pl.pallas_call(
        body,
        grid=(grid,),
        in_specs=[
            pl.BlockSpec((2, rq, 128), lambda i: (0, i, 0)),
            pl.BlockSpec((rq, 128), lambda i: (i, 0)),
            pl.BlockSpec((rq, 128), lambda i: (i, 0)),
            pl.BlockSpec((1, 128), lambda i: (0, 0)),
            pl.BlockSpec((1, 128), lambda i: (0, 0)),
            pl.BlockSpec((1, 128), lambda i: (0, 0)),
            pl.BlockSpec((128, 8), lambda i: (0, 0)),
            pl.BlockSpec((1, 8), lambda i: (0, 0)),
        ],
        out_specs=pl.BlockSpec((rq, 8), lambda i: (i, 0)),
        out_shape=jax.ShapeDtypeStruct((N_PAD // 4, 8), jnp.float32),
    )(parts, y2p, dinvp2, b2p, g2p, be2p, wlq, blq)


def kernel(x, edge_index, W1, b1, g1, be1, W2, b2, g2, be2, Wl, bl):
    src = edge_index[0].astype(jnp.int32)
    dst = edge_index[1].astype(jnp.int32)
    e = src.shape[0]
    n_chunks = -(-e // (NW * CHUNK))
    n_chunks = -(-n_chunks // NBUF) * NBUF
    epad = NW * n_chunks * CHUNK
    pad_n = epad - e
    if pad_n:
        pad_idx = N_NODES + (jnp.arange(pad_n, dtype=jnp.int32) % PAD_ROWS)
        src = jnp.concatenate([src, pad_idx])
        dst = jnp.concatenate([dst, pad_idx])
    srcp = src.reshape(NW, n_chunks, CHUNK)
    dstp = dst.reshape(NW, n_chunks, CHUNK)

    xp = jnp.zeros((N_PAD, 128), jnp.float32).at[:N_NODES].set(x)
    x2 = xp.reshape(N_PAD // 2, 256)
    ones8 = jnp.ones((CHUNK, 8), jnp.float32)
    z8 = jnp.zeros((RPT, 8), jnp.float32)
    z64 = jnp.zeros((RPT, 64), jnp.float32)
    z32 = jnp.zeros((RPT, 32), jnp.float32)

    w1b = jnp.zeros((256, 128), jnp.float32)
    w1b = w1b.at[:128, :64].set(W1).at[128:, 64:].set(W1)
    w2b4 = jnp.zeros((256, 128), jnp.float32)
    for i in range(4):
        w2b4 = w2b4.at[i * 64:(i + 1) * 64, i * 32:(i + 1) * 32].set(W2)
    wlq = jnp.zeros((128, 8), jnp.float32)
    for i in range(4):
        wlq = wlq.at[i * 32:(i + 1) * 32, i * 2:(i + 1) * 2].set(Wl)
    b1p = jnp.tile(b1, 2).reshape(1, 128)
    g1p = jnp.tile(g1, 2).reshape(1, 128)
    be1p = jnp.tile(be1, 2).reshape(1, 128)
    b2p = jnp.tile(b2, 4).reshape(1, 128)
    g2p = jnp.tile(g2, 4).reshape(1, 128)
    be2p = jnp.tile(be2, 4).reshape(1, 128)
    blq = jnp.tile(bl, 4).reshape(1, 8)

    degp = _sc_degree(dstp, ones8, z8, n_chunks)
    deg = degp[0, :, 0] + degp[1, :, 0] + 1.0
    dinv = lax.rsqrt(deg)
    dinvp1 = jnp.repeat(dinv, 64).reshape(N_PAD // 2, 128)
    dinvp2 = jnp.repeat(dinv, 32).reshape(N_PAD // 4, 128)

    y1p = _tc_prep(x2, w1b, dinvp1)
    parts1 = _sc_aggregate(y1p, srcp, dstp, z64, n_chunks, 64)
    parts1 = parts1.reshape(NC, N_PAD // 2, 128)
    y2p = _tc_mid(parts1, y1p, dinvp1, dinvp2, b1p, g1p, be1p, w2b4)
    parts2 = _sc_aggregate(y2p, srcp, dstp, z32, n_chunks, 32)
    parts2 = parts2.reshape(NC, N_PAD // 4, 128)
    out = _tc_out(parts2, y2p, dinvp2, b2p, g2p, be2p, wlq, blq)
    return out.reshape(N_PAD, 2)[:N_NODES]


# speedup vs baseline: 56.8114x; 1.0634x over previous
import functools

import jax
import jax.numpy as jnp
from jax import lax
from jax.experimental import pallas as pl
from jax.experimental.pallas import tpu as pltpu
from jax.experimental.pallas import tpu_sc as plsc

N_NODES = 10000
N_PAD = 10240
PAD_ROWS = N_PAD - N_NODES
NC = 2
NS = 16
NW = NC * NS
CHUNK = 125
RPT = N_PAD // NS
EPS = 1e-5
ROW_BLK = 1280
NBUF = 8


def _worker(c, s):
    return c * NS + s


def _sc_degree(edges, ones, z, n_chunks):
    mesh = plsc.VectorSubcoreMesh(core_axis_name="c", subcore_axis_name="s")

    @functools.partial(
        pl.kernel,
        out_type=jax.ShapeDtypeStruct((NC, N_PAD, 8), jnp.float32),
        mesh=mesh,
        scratch_types=[
            pltpu.VMEM((n_chunks, CHUNK), jnp.int32),
            pltpu.VMEM((CHUNK, 8), jnp.float32),
            pltpu.VMEM_SHARED((N_PAD, 8), jnp.float32),
            pltpu.SemaphoreType.DMA,
        ],
        compiler_params=pltpu.CompilerParams(use_tc_tiling_on_sc=False),
    )
    def deg_kernel(e_hbm, ones_hbm, z_hbm, out_hbm, idx_v, ones_v, hist_sh,
                   sem):
        c = lax.axis_index("c")
        s = lax.axis_index("s")
        w = _worker(c, s)
        pltpu.sync_copy(e_hbm.at[1, w], idx_v)
        pltpu.sync_copy(ones_hbm, ones_v)
        pltpu.sync_copy(z_hbm, hist_sh.at[pl.ds(s * RPT, RPT)])
        plsc.subcore_barrier()

        def body(j, carry):
            pltpu.async_copy(ones_v, hist_sh.at[idx_v.at[j]], sem, add=True)
            return carry

        lax.fori_loop(0, n_chunks, body, 0)

        def drain(j, carry):
            pltpu.make_async_copy(ones_v, hist_sh.at[idx_v.at[0]], sem).wait()
            return carry

        lax.fori_loop(0, n_chunks, drain, 0)
        plsc.subcore_barrier()
        pltpu.sync_copy(hist_sh.at[pl.ds(s * RPT, RPT)],
                        out_hbm.at[c, pl.ds(s * RPT, RPT)])

    return deg_kernel(edges, ones, z)


def _sc_aggregate(yp, edges, z, n_chunks, d):
    mesh = plsc.VectorSubcoreMesh(core_axis_name="c", subcore_axis_name="s")

    @functools.partial(
        pl.kernel,
        out_type=jax.ShapeDtypeStruct((NC, N_PAD, d), jnp.float32),
        mesh=mesh,
        scratch_types=[
            pltpu.VMEM((n_chunks, CHUNK), jnp.int32),
            pltpu.VMEM((n_chunks, CHUNK), jnp.int32),
            pltpu.VMEM((NBUF, CHUNK, d), jnp.float32),
            pltpu.VMEM_SHARED((N_PAD, d), jnp.float32),
        ] + [pltpu.SemaphoreType.DMA] * NBUF,
        compiler_params=pltpu.CompilerParams(use_tc_tiling_on_sc=False),
    )
    def agg_kernel(y_hbm, e_hbm, z_hbm, out_hbm,
                   src_v, dst_v, rows_v, acc_sh, *sems):
        c = lax.axis_index("c")
        s = lax.axis_index("s")
        w = _worker(c, s)
        y_flat = y_hbm
        pltpu.sync_copy(e_hbm.at[0, w], src_v)
        pltpu.sync_copy(e_hbm.at[1, w], dst_v)
        for b in range(NBUF):
            pltpu.async_copy(y_flat.at[src_v.at[b]], rows_v.at[b], sems[b])
        pltpu.sync_copy(z_hbm, acc_sh.at[pl.ds(s * RPT, RPT)])
        plsc.subcore_barrier()

        def body(g, carry):
            base = g * NBUF
            for b in range(NBUF):
                j = base + b
                pltpu.make_async_copy(y_flat.at[src_v.at[j]], rows_v.at[b],
                                      sems[b]).wait()
                pltpu.sync_copy(rows_v.at[b], acc_sh.at[dst_v.at[j]],
                                add=True)
                pltpu.async_copy(y_flat.at[src_v.at[j + NBUF]], rows_v.at[b],
                                 sems[b])
            return carry

        lax.fori_loop(0, n_chunks // NBUF - 1, body, 0)
        for b in range(NBUF):
            j = n_chunks - NBUF + b
            pltpu.make_async_copy(y_flat.at[src_v.at[j]], rows_v.at[b],
                                  sems[b]).wait()
            pltpu.sync_copy(rows_v.at[b], acc_sh.at[dst_v.at[j]], add=True)
        plsc.subcore_barrier()
        pltpu.sync_copy(acc_sh.at[pl.ds(s * RPT, RPT)],
                        out_hbm.at[c, pl.ds(s * RPT, RPT)])

    return agg_kernel(yp.reshape(N_PAD, d), edges, z)


def _tc_prep(x, w1b):
    def body(x_ref, w_ref, y_ref):
        xm = x_ref[...].reshape(x_ref.shape[0] // 2, 256)
        y_ref[...] = jnp.dot(xm, w_ref[...],
                             preferred_element_type=jnp.float32)

    grid = N_PAD // ROW_BLK
    rb = ROW_BLK // 2
    return pl.pallas_call(
        body,
        grid=(grid,),
        in_specs=[
            pl.BlockSpec((ROW_BLK, 128), lambda i: (i, 0)),
            pl.BlockSpec((256, 128), lambda i: (0, 0)),
        ],
        out_specs=pl.BlockSpec((rb, 128), lambda i: (i, 0)),
        out_shape=jax.ShapeDtypeStruct((N_PAD // 2, 128), jnp.float32),
    )(x, w1b)


def _tc_mid(parts, y1p, dinvp1, b1p, g1p, be1p, w2b4):
    def body(p_ref, y_ref, d1_ref, b_ref, g_ref, be_ref, w_ref, o_ref):
        d1 = d1_ref[...]
        pre = (p_ref[0] + p_ref[1] + y_ref[...]) * d1 + b_ref[...]
        bns = g_ref[...] * (1.0 / jnp.sqrt(1.0 + EPS))
        h = jnp.maximum(pre * bns + be_ref[...], 0.0)
        hm = (h * d1).reshape(h.shape[0] // 2, 256)
        o_ref[...] = jnp.dot(hm, w_ref[...],
                             preferred_element_type=jnp.float32)

    grid = N_PAD // ROW_BLK
    rb = ROW_BLK // 2
    rq = ROW_BLK // 4
    return pl.pallas_call(
        body,
        grid=(grid,),
        in_specs=[
            pl.BlockSpec((2, rb, 128), lambda i: (0, i, 0)),
            pl.BlockSpec((rb, 128), lambda i: (i, 0)),
            pl.BlockSpec((rb, 128), lambda i: (i, 0)),
            pl.BlockSpec((1, 128), lambda i: (0, 0)),
            pl.BlockSpec((1, 128), lambda i: (0, 0)),
            pl.BlockSpec((1, 128), lambda i: (0, 0)),
            pl.BlockSpec((256, 128), lambda i: (0, 0)),
        ],
        out_specs=pl.BlockSpec((rq, 128), lambda i: (i, 0)),
        out_shape=jax.ShapeDtypeStruct((N_PAD // 4, 128), jnp.float32),
    )(parts, y1p, dinvp1, b1p, g1p, be1p, w2b4)


def _tc_out(parts, y2p, dinvp2, b2p, g2p, be2p, wlq, blq):
    def body(p_ref, y_ref, d_ref, b_ref, g_ref, be_ref, w_ref, bl_ref,
             o_ref):
        pre = (p_ref[0] + p_ref[1] + y_ref[...]) * d_ref[...] + b_ref[...]
        bns = g_ref[...] * (1.0 / jnp.sqrt(1.0 + EPS))
        h = jnp.maximum(pre * bns + be_ref[...], 0.0)
        o_ref[...] = jnp.dot(h, w_ref[...],
                             preferred_element_type=jnp.float32) + bl_ref[...]

    grid = N_PAD // ROW_BLK
    rq = ROW_BLK // 4
    return pl.pallas_call(
        body,
        grid=(grid,),
        in_specs=[
            pl.BlockSpec((2, rq, 128), lambda i: (0, i, 0)),
            pl.BlockSpec((rq, 128), lambda i: (i, 0)),
            pl.BlockSpec((rq, 128), lambda i: (i, 0)),
            pl.BlockSpec((1, 128), lambda i: (0, 0)),
            pl.BlockSpec((1, 128), lambda i: (0, 0)),
            pl.BlockSpec((1, 128), lambda i: (0, 0)),
            pl.BlockSpec((128, 8), lambda i: (0, 0)),
            pl.BlockSpec((1, 8), lambda i: (0, 0)),
        ],
        out_specs=pl.BlockSpec((rq, 8), lambda i: (i, 0)),
        out_shape=jax.ShapeDtypeStruct((N_PAD // 4, 8), jnp.float32),
    )(parts, y2p, dinvp2, b2p, g2p, be2p, wlq, blq)


def kernel(x, edge_index, W1, b1, g1, be1, W2, b2, g2, be2, Wl, bl):
    src = edge_index[0].astype(jnp.int32)
    dst = edge_index[1].astype(jnp.int32)
    e = src.shape[0]
    n_chunks = -(-e // (NW * CHUNK))
    n_chunks = -(-n_chunks // NBUF) * NBUF
    epad = NW * n_chunks * CHUNK
    pad_n = epad - e
    if pad_n:
        pad_idx = N_NODES + (jnp.arange(pad_n, dtype=jnp.int32) % PAD_ROWS)
        src = jnp.concatenate([src, pad_idx])
        dst = jnp.concatenate([dst, pad_idx])
        edges = jnp.stack([src, dst]).reshape(2, NW, n_chunks, CHUNK)
    else:
        edges = jnp.stack([src, dst]).reshape(2, NW, n_chunks, CHUNK)

    xp = jnp.zeros((N_PAD, 128), jnp.float32).at[:N_NODES].set(x)
    ones8 = jnp.ones((CHUNK, 8), jnp.float32)
    z8 = jnp.zeros((RPT, 8), jnp.float32)
    z64 = jnp.zeros((RPT, 64), jnp.float32)
    z32 = jnp.zeros((RPT, 32), jnp.float32)

    w1b = jnp.zeros((256, 128), jnp.float32)
    w1b = w1b.at[:128, :64].set(W1).at[128:, 64:].set(W1)
    w2b4 = jnp.zeros((256, 128), jnp.float32)
    for i in range(4):
        w2b4 = w2b4.at[i * 64:(i + 1) * 64, i * 32:(i + 1) * 32].set(W2)
    wlq = jnp.zeros((128, 8), jnp.float32)
    for i in range(4):
        wlq = wlq.at[i * 32:(i + 1) * 32, i * 2:(i + 1) * 2].set(Wl)
    b1p = jnp.tile(b1, 2).reshape(1, 128)
    g1p = jnp.tile(g1, 2).reshape(1, 128)
    be1p = jnp.tile(be1, 2).reshape(1, 128)
    b2p = jnp.tile(b2, 4).reshape(1, 128)
    g2p = jnp.tile(g2, 4).reshape(1, 128)
    be2p = jnp.tile(be2, 4).reshape(1, 128)
    blq = jnp.tile(bl, 4).reshape(1, 8)

    degp = _sc_degree(edges, ones8, z8, n_chunks)
    xwp = _tc_prep(xp, w1b)
    deg = degp[0, :, 0] + degp[1, :, 0] + 1.0
    dinv = lax.rsqrt(deg)
    dinvp1 = jnp.repeat(dinv, 64).reshape(N_PAD // 2, 128)
    dinvp2 = jnp.repeat(dinv, 32).reshape(N_PAD // 4, 128)

    y1p = xwp * dinvp1
    parts1 = _sc_aggregate(y1p, edges, z64, n_chunks, 64)
    parts1 = parts1.reshape(NC, N_PAD // 2, 128)
    y2p = _tc_mid(parts1, y1p, dinvp1, b1p, g1p, be1p, w2b4)
    parts2 = _sc_aggregate(y2p, edges, z32, n_chunks, 32)
    parts2 = parts2.reshape(NC, N_PAD // 4, 128)
    out = _tc_out(parts2, y2p, dinvp2, b2p, g2p, be2p, wlq, blq)
    return out.reshape(N_PAD, 2)[:N_NODES]
